# baseline (speedup 1.0000x reference)
import functools

import jax
import jax.numpy as jnp
from jax import lax
from jax.experimental import pallas as pl
from jax.experimental.pallas import tpu as pltpu
from jax.experimental.pallas import tpu_sc as plsc

T, D, EK, S = 2048, 1024, 128, 32
TB = 256

_info = plsc.get_sparse_core_info()
_NC, _NS, _L = _info.num_cores, _info.num_subcores, _info.num_lanes
_NW = _NC * _NS
_TPW = T // _NW


@functools.partial(
    pl.kernel,
    mesh=plsc.VectorSubcoreMesh(core_axis_name="c", subcore_axis_name="s"),
    out_type=jax.ShapeDtypeStruct((T * EK,), jnp.float32),
    scratch_types=[
        pltpu.VMEM((S, _TPW), jnp.int32),
        pltpu.VMEM((_TPW * EK,), jnp.float32),
    ],
    compiler_params=pltpu.CompilerParams(needs_layout_passes=False),
)
def _hist_sc(idxt_hbm, out_hbm, idx_v, cnt_v):
    wid = lax.axis_index("s") * _NC + lax.axis_index("c")
    pltpu.sync_copy(idxt_hbm.at[wid], idx_v)

    zeros = jnp.zeros((_L,), jnp.float32)
    for i in range(_TPW * EK // _L):
        cnt_v[pl.ds(i * _L, _L)] = zeros

    ones = jnp.ones((_L,), jnp.float32)
    row_iota = lax.iota(jnp.int32, _L)
    for g in range(_TPW // _L):
        row_off = (row_iota + g * _L) * EK
        for s in range(S):
            cols = idx_v[s, pl.ds(g * _L, _L)]
            plsc.addupdate_scatter(cnt_v, [row_off + cols], ones)

    pltpu.sync_copy(cnt_v, out_hbm.at[pl.ds(wid * _TPW * EK, _TPW * EK)])


def _p_body(x_ref, svh_ref, p_ref):
    p_ref[...] = lax.dot_general(x_ref[...], svh_ref[...],
                                 (((1,), (1,)), ((), ())),
                                 preferred_element_type=jnp.float32)


def _out_body(p_ref, c_ref, u_ref, o_ref):
    scaled = p_ref[...] * c_ref[...]
    o_ref[...] = lax.dot_general(scaled, u_ref[...], (((1,), (1,)), ((), ())),
                                 preferred_element_type=jnp.float32)


@jax.jit
def _run(x, index, u, svh):
    idxt = jnp.transpose(index).reshape(S, _NW, _TPW).transpose(1, 0, 2)
    counts = _hist_sc(idxt).reshape(T, EK)
    p = pl.pallas_call(
        _p_body,
        grid=(T // TB,),
        in_specs=[
            pl.BlockSpec((TB, D), lambda i: (i, 0)),
            pl.BlockSpec((EK, D), lambda i: (0, 0)),
        ],
        out_specs=pl.BlockSpec((TB, EK), lambda i: (i, 0)),
        out_shape=jax.ShapeDtypeStruct((T, EK), jnp.float32),
    )(x, svh)
    return pl.pallas_call(
        _out_body,
        grid=(T // TB,),
        in_specs=[
            pl.BlockSpec((TB, EK), lambda i: (i, 0)),
            pl.BlockSpec((TB, EK), lambda i: (i, 0)),
            pl.BlockSpec((D, EK), lambda i: (0, 0)),
        ],
        out_specs=pl.BlockSpec((TB, D), lambda i: (i, 0)),
        out_shape=jax.ShapeDtypeStruct((T, D), jnp.float32),
    )(p, counts, u)


def kernel(x, routing_weights, index, u, svh):
    del routing_weights
    return _run(x, index, u, svh)


# speedup vs baseline: 3.0813x; 3.0813x over previous
import functools

import jax
import jax.numpy as jnp
from jax import lax
from jax.experimental import pallas as pl

T, D, EK, S = 2048, 1024, 128, 32
TB = 512


def _body(x_ref, idxt_ref, u_ref, svh_ref, o_ref):
    p_t = lax.dot_general(svh_ref[...], x_ref[...], (((1,), (1,)), ((), ())),
                          preferred_element_type=jnp.float32)
    jota = lax.broadcasted_iota(jnp.int32, (EK, TB), 0)
    cnt_t = jnp.zeros((EK, TB), jnp.float32)
    for s in range(S):
        cnt_t = cnt_t + (idxt_ref[s:s + 1, :] == jota).astype(jnp.float32)
    scaled_t = p_t * cnt_t
    o_ref[...] = lax.dot_general(scaled_t, u_ref[...], (((0,), (1,)), ((), ())),
                                 preferred_element_type=jnp.float32)


@jax.jit
def _run(x, index, u, svh):
    idxt = index.T
    return pl.pallas_call(
        _body,
        grid=(T // TB,),
        in_specs=[
            pl.BlockSpec((TB, D), lambda i: (i, 0)),
            pl.BlockSpec((S, TB), lambda i: (0, i)),
            pl.BlockSpec((D, EK), lambda i: (0, 0)),
            pl.BlockSpec((EK, D), lambda i: (0, 0)),
        ],
        out_specs=pl.BlockSpec((TB, D), lambda i: (i, 0)),
        out_shape=jax.ShapeDtypeStruct((T, D), jnp.float32),
    )(x, idxt, u, svh)


def kernel(x, routing_weights, index, u, svh):
    del routing_weights
    return _run(x, index, u, svh)


# speedup vs baseline: 3.1020x; 1.0067x over previous
import functools

import jax
import jax.numpy as jnp
from jax import lax
from jax.experimental import pallas as pl

T, D, EK, S = 2048, 1024, 128, 32
TB = 1024


def _body(x_ref, idxt_ref, u_ref, svh_ref, o_ref):
    p_t = lax.dot_general(svh_ref[...], x_ref[...], (((1,), (1,)), ((), ())),
                          preferred_element_type=jnp.float32)
    jota = lax.broadcasted_iota(jnp.int32, (EK, TB), 0)
    cnt_t = jnp.zeros((EK, TB), jnp.float32)
    for s in range(S):
        cnt_t = cnt_t + (idxt_ref[s:s + 1, :] == jota).astype(jnp.float32)
    scaled_t = p_t * cnt_t
    o_ref[...] = lax.dot_general(scaled_t, u_ref[...], (((0,), (1,)), ((), ())),
                                 preferred_element_type=jnp.float32)


@jax.jit
def _run(x, index, u, svh):
    idxt = index.T
    return pl.pallas_call(
        _body,
        grid=(T // TB,),
        in_specs=[
            pl.BlockSpec((TB, D), lambda i: (i, 0)),
            pl.BlockSpec((S, TB), lambda i: (0, i)),
            pl.BlockSpec((D, EK), lambda i: (0, 0)),
            pl.BlockSpec((EK, D), lambda i: (0, 0)),
        ],
        out_specs=pl.BlockSpec((TB, D), lambda i: (i, 0)),
        out_shape=jax.ShapeDtypeStruct((T, D), jnp.float32),
    )(x, idxt, u, svh)


def kernel(x, routing_weights, index, u, svh):
    del routing_weights
    return _run(x, index, u, svh)
